# Initial kernel scaffold; baseline (speedup 1.0000x reference)
import jax
import jax.numpy as jnp
from jax.experimental import pallas as pl


def kernel(x, edge_index, batch, W1, b1, W2, b2, bn_gamma, bn_beta, lin_W, lin_b):
    raise NotImplementedError("write your pallas kernel here")


# speedup vs baseline: 27.4082x; 27.4082x over previous
import functools

import jax
import jax.numpy as jnp
from jax import lax
from jax.experimental import pallas as pl
from jax.experimental.pallas import tpu as pltpu
from jax.experimental.pallas import tpu_sc as plsc

N = 10000
NP = 10240
D = 128
G = 64
E = 320000
EPS = 1e-5
NC = 2
NS = 16
CHUNK = 128
CPT = 80
EPAD = NC * NS * CPT * CHUNK
RPT = NP // NS
R = 1024

_mesh = plsc.VectorSubcoreMesh(core_axis_name="c", subcore_axis_name="s")


@functools.partial(
    pl.kernel,
    out_type=jax.ShapeDtypeStruct((NC, NP), jnp.float32),
    mesh=_mesh,
    scratch_types=[
        pltpu.VMEM_SHARED((NP,), jnp.float32),
        pltpu.VMEM((CPT, CHUNK), jnp.int32),
        pltpu.VMEM((CHUNK,), jnp.float32),
        pltpu.VMEM((RPT,), jnp.float32),
        pltpu.SemaphoreType.DMA,
    ],
)
def _sc_degree(dstr, out, dacc, dst_v, ones_v, z_v, sem):
    c = lax.axis_index("c")
    s = lax.axis_index("s")
    r0 = s * RPT
    pltpu.sync_copy(dstr.at[c, s], dst_v)
    for i in range(CHUNK // 16):
        ones_v[pl.ds(i * 16, 16)] = jnp.ones((16,), jnp.float32)

    def zfill(i, carry):
        z_v[pl.ds(i * 16, 16)] = jnp.zeros((16,), jnp.float32)
        return carry

    lax.fori_loop(0, RPT // 16, zfill, 0)
    pltpu.sync_copy(z_v, dacc.at[pl.ds(r0, RPT)])
    plsc.subcore_barrier()

    FIRE = 8

    def group(g, carry):
        j0 = g * FIRE
        for b in range(FIRE):
            pltpu.async_copy(ones_v, dacc.at[dst_v.at[j0 + b]], sem, add=True)
        for b in range(FIRE):
            pltpu.make_async_copy(ones_v, dacc.at[dst_v.at[j0 + b]], sem).wait()
        return carry

    lax.fori_loop(0, CPT // FIRE, group, 0)
    plsc.subcore_barrier()
    pltpu.sync_copy(dacc.at[pl.ds(r0, RPT)], out.at[c, pl.ds(r0, RPT)])


GB = 8
NGRP = CPT // GB


@functools.partial(
    pl.kernel,
    out_type=jax.ShapeDtypeStruct((NC, NP, D), jnp.float32),
    mesh=_mesh,
    scratch_types=[
        pltpu.VMEM_SHARED((NP, D), jnp.float32),
        pltpu.VMEM((2, GB, CHUNK), jnp.int32),
        pltpu.VMEM((2, GB, CHUNK), jnp.int32),
        pltpu.VMEM((CHUNK, D), jnp.float32),
        pltpu.VMEM((CHUNK, D), jnp.float32),
        pltpu.SemaphoreType.DMA,
        pltpu.SemaphoreType.DMA,
        pltpu.SemaphoreType.DMA,
    ],
)
def _sc_aggregate(hp, srcr, dstr, out, acc, sidx, didx, buf0, buf1, gsem, ssem, isem):
    c = lax.axis_index("c")
    s = lax.axis_index("s")
    r0 = s * RPT
    pltpu.sync_copy(srcr.at[c, s, pl.ds(0, GB)], sidx.at[0])
    pltpu.sync_copy(dstr.at[c, s, pl.ds(0, GB)], didx.at[0])
    pltpu.sync_copy(hp.at[pl.ds(r0, RPT)], acc.at[pl.ds(r0, RPT)])
    plsc.subcore_barrier()

    bufs = (buf0, buf1)

    def group(g, carry):
        sl = lax.rem(g, 2)
        nsl = 1 - sl

        @pl.when(g + 1 < NGRP)
        def _prefetch_idx():
            pltpu.async_copy(
                srcr.at[c, s, pl.ds((g + 1) * GB, GB)], sidx.at[nsl], isem
            )
            pltpu.async_copy(
                dstr.at[c, s, pl.ds((g + 1) * GB, GB)], didx.at[nsl], isem
            )

        pltpu.async_copy(hp.at[sidx.at[sl, 0]], buf0, gsem)
        for j in range(GB):
            b = j % 2
            pltpu.make_async_copy(hp.at[sidx.at[sl, j]], bufs[b], gsem).wait()
            pltpu.async_copy(bufs[b], acc.at[didx.at[sl, j]], ssem, add=True)
            if j > 0:
                pltpu.make_async_copy(
                    bufs[1 - b], acc.at[didx.at[sl, j - 1]], ssem
                ).wait()
            if j + 1 < GB:
                pltpu.async_copy(hp.at[sidx.at[sl, j + 1]], bufs[1 - b], gsem)
        pltpu.make_async_copy(bufs[1], acc.at[didx.at[sl, GB - 1]], ssem).wait()

        @pl.when(g + 1 < NGRP)
        def _wait_idx():
            pltpu.make_async_copy(
                srcr.at[c, s, pl.ds((g + 1) * GB, GB)], sidx.at[nsl], isem
            ).wait()
            pltpu.make_async_copy(
                dstr.at[c, s, pl.ds((g + 1) * GB, GB)], didx.at[nsl], isem
            ).wait()

        return carry

    lax.fori_loop(0, NGRP, group, 0)
    plsc.subcore_barrier()
    pltpu.sync_copy(acc.at[pl.ds(r0, RPT)], out.at[c, pl.ds(r0, RPT)])


def _tc1_body(xp, w1, deg, hp1, dinv):
    dg = deg[0] + deg[1] + 1.0
    di = lax.rsqrt(dg)
    dinv[...] = di
    t0 = jnp.dot(xp[...], w1[...], preferred_element_type=jnp.float32)
    hp1[...] = t0 * di


def _tc_scale_in(xp, w1, degr):
    return pl.pallas_call(
        _tc1_body,
        grid=(NP // R,),
        in_specs=[
            pl.BlockSpec((R, D), lambda i: (i, 0)),
            pl.BlockSpec((D, D), lambda i: (0, 0)),
            pl.BlockSpec((NC, R, 1), lambda i: (0, i, 0)),
        ],
        out_specs=[
            pl.BlockSpec((R, D), lambda i: (i, 0)),
            pl.BlockSpec((R, 1), lambda i: (i, 0)),
        ],
        out_shape=[
            jax.ShapeDtypeStruct((NP, D), jnp.float32),
            jax.ShapeDtypeStruct((NP, 1), jnp.float32),
        ],
    )(xp, w1, degr)


def _tc3a_body(agg, hp1, dinv, b1, h_out, stats):
    i = pl.program_id(0)
    h = (agg[0] + agg[1] - hp1[...]) * dinv[...] + b1[...]
    h_out[...] = h
    rows = lax.broadcasted_iota(jnp.int32, (R, 1), 0) + i * R
    m = (rows < N).astype(jnp.float32)
    hm = h * m

    @pl.when(i == 0)
    def _():
        stats[...] = jnp.zeros_like(stats)

    stats[0:1, :] += jnp.sum(hm, axis=0, keepdims=True)
    stats[1:2, :] += jnp.sum(hm * h, axis=0, keepdims=True)


def _tc_stats(agg1, hp1, dinv, b1):
    return pl.pallas_call(
        _tc3a_body,
        grid=(NP // R,),
        in_specs=[
            pl.BlockSpec((NC, R, D), lambda i: (0, i, 0)),
            pl.BlockSpec((R, D), lambda i: (i, 0)),
            pl.BlockSpec((R, 1), lambda i: (i, 0)),
            pl.BlockSpec((D,), lambda i: (0,)),
        ],
        out_specs=[
            pl.BlockSpec((R, D), lambda i: (i, 0)),
            pl.BlockSpec((2, D), lambda i: (0, 0)),
        ],
        out_shape=[
            jax.ShapeDtypeStruct((NP, D), jnp.float32),
            jax.ShapeDtypeStruct((2, D), jnp.float32),
        ],
    )(agg1, hp1, dinv, b1)


def _tc3b_body(h, stats, gamma, beta, w2, dinv, hp2):
    mean = stats[0:1, :] / N
    var = stats[1:2, :] / N - mean * mean
    inv = lax.rsqrt(var + EPS)
    hb = (h[...] - mean) * inv * gamma[...] + beta[...]
    hb = jnp.maximum(hb, 0.0)
    hp2[...] = jnp.dot(hb, w2[...], preferred_element_type=jnp.float32) * dinv[...]


def _tc_bn_mm(h1, stats, gamma, beta, w2, dinv):
    return pl.pallas_call(
        _tc3b_body,
        grid=(NP // R,),
        in_specs=[
            pl.BlockSpec((R, D), lambda i: (i, 0)),
            pl.BlockSpec((2, D), lambda i: (0, 0)),
            pl.BlockSpec((D,), lambda i: (0,)),
            pl.BlockSpec((D,), lambda i: (0,)),
            pl.BlockSpec((D, D), lambda i: (0, 0)),
            pl.BlockSpec((R, 1), lambda i: (i, 0)),
        ],
        out_specs=pl.BlockSpec((R, D), lambda i: (i, 0)),
        out_shape=jax.ShapeDtypeStruct((NP, D), jnp.float32),
    )(h1, stats, gamma, beta, w2, dinv)


def _tc4_body(agg, hp2, dinv, b2, batchp, linw, linb, out, sums, cnts):
    i = pl.program_id(0)
    h2 = (agg[0] + agg[1] - hp2[...]) * dinv[...] + b2[...]
    oh = (batchp[...] == lax.broadcasted_iota(jnp.int32, (R, G), 1)).astype(
        jnp.float32
    )

    @pl.when(i == 0)
    def _():
        sums[...] = jnp.zeros_like(sums)
        cnts[...] = jnp.zeros_like(cnts)

    sums[...] += lax.dot_general(
        oh, h2, (((0,), (0,)), ((), ())), preferred_element_type=jnp.float32
    )
    cnts[...] += lax.dot_general(
        oh,
        jnp.ones((R, 1), jnp.float32),
        (((0,), (0,)), ((), ())),
        preferred_element_type=jnp.float32,
    )

    @pl.when(i == pl.num_programs(0) - 1)
    def _():
        pooled = sums[...] / jnp.maximum(cnts[...], 1.0)
        out[...] = (
            jnp.dot(pooled, linw[...], preferred_element_type=jnp.float32)
            + linb[...]
        )


def _tc_final(agg2, hp2, dinv, b2, batchp, lin_w, lin_b):
    return pl.pallas_call(
        _tc4_body,
        grid=(NP // R,),
        in_specs=[
            pl.BlockSpec((NC, R, D), lambda i: (0, i, 0)),
            pl.BlockSpec((R, D), lambda i: (i, 0)),
            pl.BlockSpec((R, 1), lambda i: (i, 0)),
            pl.BlockSpec((D,), lambda i: (0,)),
            pl.BlockSpec((R, 1), lambda i: (i, 0)),
            pl.BlockSpec((D, D), lambda i: (0, 0)),
            pl.BlockSpec((D,), lambda i: (0,)),
        ],
        out_specs=pl.BlockSpec((G, D), lambda i: (0, 0)),
        out_shape=jax.ShapeDtypeStruct((G, D), jnp.float32),
        scratch_shapes=[
            pltpu.VMEM((G, D), jnp.float32),
            pltpu.VMEM((G, 1), jnp.float32),
        ],
    )(agg2, hp2, dinv, b2, batchp, lin_w, lin_b)


def kernel(x, edge_index, batch, W1, b1, W2, b2, bn_gamma, bn_beta, lin_W, lin_b):
    src = edge_index[0].astype(jnp.int32)
    dst = edge_index[1].astype(jnp.int32)
    padi = N + (jnp.arange(EPAD - E, dtype=jnp.int32) % (NP - N))
    srcp = jnp.concatenate([src, padi]).reshape(NC, NS, CPT, CHUNK)
    dstp = jnp.concatenate([dst, padi]).reshape(NC, NS, CPT, CHUNK)
    xp = jnp.zeros((NP, D), jnp.float32).at[:N].set(x)
    batchp = jnp.concatenate(
        [batch.astype(jnp.int32), jnp.full((NP - N,), G, jnp.int32)]
    ).reshape(NP, 1)

    deg = _sc_degree(dstp)
    degr = deg.reshape(NC, NP, 1)
    hp1, dinv = _tc_scale_in(xp, W1, degr)
    agg1 = _sc_aggregate(hp1, srcp, dstp)
    h1, stats = _tc_stats(agg1, hp1, dinv, b1)
    hp2 = _tc_bn_mm(h1, stats, bn_gamma, bn_beta, W2, dinv)
    agg2 = _sc_aggregate(hp2, srcp, dstp)
    return _tc_final(agg2, hp2, dinv, b2, batchp, lin_W, lin_b)


# speedup vs baseline: 34.3596x; 1.2536x over previous
import functools

import jax
import jax.numpy as jnp
from jax import lax
from jax.experimental import pallas as pl
from jax.experimental.pallas import tpu as pltpu
from jax.experimental.pallas import tpu_sc as plsc

N = 10000
NP = 10240
D = 128
G = 64
E = 320000
EPS = 1e-5
NC = 2
NS = 16
CHUNK = 112
CPT = 92
EPAD = NC * NS * CPT * CHUNK
RPT = NP // NS
R = 1024

_mesh = plsc.VectorSubcoreMesh(core_axis_name="c", subcore_axis_name="s")


@functools.partial(
    pl.kernel,
    out_type=jax.ShapeDtypeStruct((NC, NP), jnp.float32),
    mesh=_mesh,
    scratch_types=[
        pltpu.VMEM_SHARED((NP,), jnp.float32),
        pltpu.VMEM((CPT, CHUNK), jnp.int32),
        pltpu.VMEM((CHUNK,), jnp.float32),
        pltpu.VMEM((RPT,), jnp.float32),
        pltpu.SemaphoreType.DMA,
    ],
)
def _sc_degree(dstr, out, dacc, dst_v, ones_v, z_v, sem):
    c = lax.axis_index("c")
    s = lax.axis_index("s")
    r0 = s * RPT
    pltpu.sync_copy(dstr.at[c, s], dst_v)
    for i in range(CHUNK // 16):
        ones_v[pl.ds(i * 16, 16)] = jnp.ones((16,), jnp.float32)

    def zfill(i, carry):
        z_v[pl.ds(i * 16, 16)] = jnp.zeros((16,), jnp.float32)
        return carry

    lax.fori_loop(0, RPT // 16, zfill, 0)
    pltpu.sync_copy(z_v, dacc.at[pl.ds(r0, RPT)])
    plsc.subcore_barrier()

    FIRE = 4

    def group(g, carry):
        j0 = g * FIRE
        for b in range(FIRE):
            pltpu.async_copy(ones_v, dacc.at[dst_v.at[j0 + b]], sem, add=True)
        for b in range(FIRE):
            pltpu.make_async_copy(ones_v, dacc.at[dst_v.at[j0 + b]], sem).wait()
        return carry

    lax.fori_loop(0, CPT // FIRE, group, 0)
    plsc.subcore_barrier()
    pltpu.sync_copy(dacc.at[pl.ds(r0, RPT)], out.at[c, pl.ds(r0, RPT)])


NBUF = 3
GB = 4
NGRP = CPT // GB


@functools.partial(
    pl.kernel,
    out_type=jax.ShapeDtypeStruct((NC, NP, D), jnp.float32),
    mesh=_mesh,
    scratch_types=[
        pltpu.VMEM_SHARED((NP, D), jnp.float32),
        pltpu.VMEM((2, GB, CHUNK), jnp.int32),
        pltpu.VMEM((2, GB, CHUNK), jnp.int32),
        pltpu.VMEM((NBUF, CHUNK, D), jnp.float32),
        pltpu.SemaphoreType.DMA,
        pltpu.SemaphoreType.DMA,
        pltpu.SemaphoreType.DMA,
    ],
)
def _sc_aggregate(hp, srcr, dstr, out, acc, sidx, didx, bufs, gsem, ssem, isem):
    c = lax.axis_index("c")
    s = lax.axis_index("s")
    r0 = s * RPT
    pltpu.sync_copy(srcr.at[c, s, pl.ds(0, GB)], sidx.at[0])
    pltpu.sync_copy(dstr.at[c, s, pl.ds(0, GB)], didx.at[0])
    pltpu.sync_copy(hp.at[pl.ds(r0, RPT)], acc.at[pl.ds(r0, RPT)])
    plsc.subcore_barrier()

    pltpu.async_copy(hp.at[sidx.at[0, 0]], bufs.at[0], gsem)
    pltpu.async_copy(hp.at[sidx.at[0, 1]], bufs.at[1], gsem)

    def step(j, carry):
        g = j // GB
        p = j % GB
        sl = g % 2
        b = j % NBUF
        jm = j - 1
        slm = (jm // GB) % 2
        pm = jm % GB
        bm = jm % NBUF
        j2 = j + 2
        sl2 = (j2 // GB) % 2
        p2 = j2 % GB
        b2 = j2 % NBUF

        pltpu.make_async_copy(hp.at[sidx.at[sl, p]], bufs.at[b], gsem).wait()
        pltpu.async_copy(bufs.at[b], acc.at[didx.at[sl, p]], ssem, add=True)

        @pl.when(j > 0)
        def _wait_prev_scatter():
            pltpu.make_async_copy(
                bufs.at[bm], acc.at[didx.at[slm, pm]], ssem
            ).wait()

        @pl.when(jnp.logical_and(p == 0, g + 1 < NGRP))
        def _prefetch_idx():
            pltpu.async_copy(
                srcr.at[c, s, pl.ds((g + 1) * GB, GB)], sidx.at[1 - sl], isem
            )
            pltpu.async_copy(
                dstr.at[c, s, pl.ds((g + 1) * GB, GB)], didx.at[1 - sl], isem
            )

        @pl.when(jnp.logical_and(p == GB - 2, g + 1 < NGRP))
        def _wait_idx():
            pltpu.make_async_copy(
                srcr.at[c, s, pl.ds((g + 1) * GB, GB)], sidx.at[1 - sl], isem
            ).wait()
            pltpu.make_async_copy(
                dstr.at[c, s, pl.ds((g + 1) * GB, GB)], didx.at[1 - sl], isem
            ).wait()

        @pl.when(j2 < CPT)
        def _next_gather():
            pltpu.async_copy(hp.at[sidx.at[sl2, p2]], bufs.at[b2], gsem)

        return carry

    lax.fori_loop(0, CPT, step, 0)
    pltpu.make_async_copy(
        bufs.at[(CPT - 1) % NBUF],
        acc.at[didx.at[((CPT - 1) // GB) % 2, (CPT - 1) % GB]],
        ssem,
    ).wait()
    plsc.subcore_barrier()
    pltpu.sync_copy(acc.at[pl.ds(r0, RPT)], out.at[c, pl.ds(r0, RPT)])


def _tc1_body(xp, w1, deg, hp1, dinv):
    dg = deg[0] + deg[1] + 1.0
    di = lax.rsqrt(dg)
    dinv[...] = di
    t0 = jnp.dot(xp[...], w1[...], preferred_element_type=jnp.float32)
    hp1[...] = t0 * di


def _tc_scale_in(xp, w1, degr):
    return pl.pallas_call(
        _tc1_body,
        grid=(NP // R,),
        in_specs=[
            pl.BlockSpec((R, D), lambda i: (i, 0)),
            pl.BlockSpec((D, D), lambda i: (0, 0)),
            pl.BlockSpec((NC, R, 1), lambda i: (0, i, 0)),
        ],
        out_specs=[
            pl.BlockSpec((R, D), lambda i: (i, 0)),
            pl.BlockSpec((R, 1), lambda i: (i, 0)),
        ],
        out_shape=[
            jax.ShapeDtypeStruct((NP, D), jnp.float32),
            jax.ShapeDtypeStruct((NP, 1), jnp.float32),
        ],
    )(xp, w1, degr)


def _tc_mid_body(agg, hp1, dinv, b1, gamma, beta, w2, hp2, stats):
    p = pl.program_id(0)
    i = pl.program_id(1)
    h = (agg[0] + agg[1] - hp1[...]) * dinv[...] + b1[...]

    @pl.when(p == 0)
    def _accumulate_stats():
        rows = lax.broadcasted_iota(jnp.int32, (R, 1), 0) + i * R
        m = (rows < N).astype(jnp.float32)
        hm = h * m

        @pl.when(i == 0)
        def _():
            stats[...] = jnp.zeros_like(stats)

        stats[0:1, :] += jnp.sum(hm, axis=0, keepdims=True)
        stats[1:2, :] += jnp.sum(hm * h, axis=0, keepdims=True)

    @pl.when(p == 1)
    def _normalize_matmul():
        mean = stats[0:1, :] / N
        var = stats[1:2, :] / N - mean * mean
        inv = lax.rsqrt(var + EPS)
        hb = (h - mean) * inv * gamma[...] + beta[...]
        hb = jnp.maximum(hb, 0.0)
        hp2[...] = (
            jnp.dot(hb, w2[...], preferred_element_type=jnp.float32) * dinv[...]
        )


def _tc_mid(agg1, hp1, dinv, b1, gamma, beta, w2):
    return pl.pallas_call(
        _tc_mid_body,
        grid=(2, NP // R),
        in_specs=[
            pl.BlockSpec((NC, R, D), lambda p, i: (0, i, 0)),
            pl.BlockSpec((R, D), lambda p, i: (i, 0)),
            pl.BlockSpec((R, 1), lambda p, i: (i, 0)),
            pl.BlockSpec((D,), lambda p, i: (0,)),
            pl.BlockSpec((D,), lambda p, i: (0,)),
            pl.BlockSpec((D,), lambda p, i: (0,)),
            pl.BlockSpec((D, D), lambda p, i: (0, 0)),
        ],
        out_specs=pl.BlockSpec((R, D), lambda p, i: (i, 0)),
        out_shape=jax.ShapeDtypeStruct((NP, D), jnp.float32),
        scratch_shapes=[pltpu.VMEM((2, D), jnp.float32)],
    )(agg1, hp1, dinv, b1, gamma, beta, w2)


def _tc4_body(agg, hp2, dinv, b2, batchp, linw, linb, out, sums, cnts):
    i = pl.program_id(0)
    h2 = (agg[0] + agg[1] - hp2[...]) * dinv[...] + b2[...]
    oh = (batchp[...] == lax.broadcasted_iota(jnp.int32, (R, G), 1)).astype(
        jnp.float32
    )

    @pl.when(i == 0)
    def _():
        sums[...] = jnp.zeros_like(sums)
        cnts[...] = jnp.zeros_like(cnts)

    sums[...] += lax.dot_general(
        oh, h2, (((0,), (0,)), ((), ())), preferred_element_type=jnp.float32
    )
    cnts[...] += lax.dot_general(
        oh,
        jnp.ones((R, 1), jnp.float32),
        (((0,), (0,)), ((), ())),
        preferred_element_type=jnp.float32,
    )

    @pl.when(i == pl.num_programs(0) - 1)
    def _():
        pooled = sums[...] / jnp.maximum(cnts[...], 1.0)
        out[...] = (
            jnp.dot(pooled, linw[...], preferred_element_type=jnp.float32)
            + linb[...]
        )


def _tc_final(agg2, hp2, dinv, b2, batchp, lin_w, lin_b):
    return pl.pallas_call(
        _tc4_body,
        grid=(NP // R,),
        in_specs=[
            pl.BlockSpec((NC, R, D), lambda i: (0, i, 0)),
            pl.BlockSpec((R, D), lambda i: (i, 0)),
            pl.BlockSpec((R, 1), lambda i: (i, 0)),
            pl.BlockSpec((D,), lambda i: (0,)),
            pl.BlockSpec((R, 1), lambda i: (i, 0)),
            pl.BlockSpec((D, D), lambda i: (0, 0)),
            pl.BlockSpec((D,), lambda i: (0,)),
        ],
        out_specs=pl.BlockSpec((G, D), lambda i: (0, 0)),
        out_shape=jax.ShapeDtypeStruct((G, D), jnp.float32),
        scratch_shapes=[
            pltpu.VMEM((G, D), jnp.float32),
            pltpu.VMEM((G, 1), jnp.float32),
        ],
    )(agg2, hp2, dinv, b2, batchp, lin_w, lin_b)


def kernel(x, edge_index, batch, W1, b1, W2, b2, bn_gamma, bn_beta, lin_W, lin_b):
    src = edge_index[0].astype(jnp.int32)
    dst = edge_index[1].astype(jnp.int32)
    padi = N + (jnp.arange(EPAD - E, dtype=jnp.int32) % (NP - N))
    srcp = jnp.concatenate([src, padi]).reshape(NC, NS, CPT, CHUNK)
    dstp = jnp.concatenate([dst, padi]).reshape(NC, NS, CPT, CHUNK)
    xp = jnp.zeros((NP, D), jnp.float32).at[:N].set(x)
    batchp = jnp.concatenate(
        [batch.astype(jnp.int32), jnp.full((NP - N,), G, jnp.int32)]
    ).reshape(NP, 1)

    deg = _sc_degree(dstp)
    degr = deg.reshape(NC, NP, 1)
    hp1, dinv = _tc_scale_in(xp, W1, degr)
    agg1 = _sc_aggregate(hp1, srcp, dstp)
    hp2 = _tc_mid(agg1, hp1, dinv, b1, bn_gamma, bn_beta, W2)
    agg2 = _sc_aggregate(hp2, srcp, dstp)
    return _tc_final(agg2, hp2, dinv, b2, batchp, lin_W, lin_b)


# speedup vs baseline: 35.1050x; 1.0217x over previous
import functools

import jax
import jax.numpy as jnp
from jax import lax
from jax.experimental import pallas as pl
from jax.experimental.pallas import tpu as pltpu
from jax.experimental.pallas import tpu_sc as plsc

N = 10000
NP = 10240
D = 128
G = 64
E = 320000
EPS = 1e-5
NC = 2
NS = 16
CHUNK = 80
CPT = 128
EPAD = NC * NS * CPT * CHUNK
RPT = NP // NS
R = 1024

_mesh = plsc.VectorSubcoreMesh(core_axis_name="c", subcore_axis_name="s")


@functools.partial(
    pl.kernel,
    out_type=jax.ShapeDtypeStruct((NC, NP), jnp.float32),
    mesh=_mesh,
    scratch_types=[
        pltpu.VMEM_SHARED((NP,), jnp.float32),
        pltpu.VMEM((CPT, CHUNK), jnp.int32),
        pltpu.VMEM((CHUNK,), jnp.float32),
        pltpu.VMEM((RPT,), jnp.float32),
        pltpu.SemaphoreType.DMA,
    ],
)
def _sc_degree(dstr, out, dacc, dst_v, ones_v, z_v, sem):
    c = lax.axis_index("c")
    s = lax.axis_index("s")
    r0 = s * RPT
    pltpu.sync_copy(dstr.at[c, s], dst_v)
    for i in range(CHUNK // 16):
        ones_v[pl.ds(i * 16, 16)] = jnp.ones((16,), jnp.float32)

    def zfill(i, carry):
        z_v[pl.ds(i * 16, 16)] = jnp.zeros((16,), jnp.float32)
        return carry

    lax.fori_loop(0, RPT // 16, zfill, 0)
    pltpu.sync_copy(z_v, dacc.at[pl.ds(r0, RPT)])
    plsc.subcore_barrier()

    FIRE = 4

    def group(g, carry):
        j0 = g * FIRE
        for b in range(FIRE):
            pltpu.async_copy(ones_v, dacc.at[dst_v.at[j0 + b]], sem, add=True)
        for b in range(FIRE):
            pltpu.make_async_copy(ones_v, dacc.at[dst_v.at[j0 + b]], sem).wait()
        return carry

    lax.fori_loop(0, CPT // FIRE, group, 0)
    plsc.subcore_barrier()
    pltpu.sync_copy(dacc.at[pl.ds(r0, RPT)], out.at[c, pl.ds(r0, RPT)])


NBUF = 4
GB = 8
NGRP = CPT // GB


@functools.partial(
    pl.kernel,
    out_type=jax.ShapeDtypeStruct((NC, NP, D), jnp.float32),
    mesh=_mesh,
    scratch_types=[
        pltpu.VMEM_SHARED((NP, D), jnp.float32),
        pltpu.VMEM((2, GB, CHUNK), jnp.int32),
        pltpu.VMEM((2, GB, CHUNK), jnp.int32),
        pltpu.VMEM((NBUF, CHUNK, D), jnp.float32),
        pltpu.SemaphoreType.DMA,
        pltpu.SemaphoreType.DMA,
        pltpu.SemaphoreType.DMA,
    ],
)
def _sc_aggregate(hp, srcr, dstr, out, acc, sidx, didx, bufs, gsem, ssem, isem):
    c = lax.axis_index("c")
    s = lax.axis_index("s")
    r0 = s * RPT
    pltpu.sync_copy(srcr.at[c, s, pl.ds(0, GB)], sidx.at[0])
    pltpu.sync_copy(dstr.at[c, s, pl.ds(0, GB)], didx.at[0])
    pltpu.sync_copy(hp.at[pl.ds(r0, RPT)], acc.at[pl.ds(r0, RPT)])
    plsc.subcore_barrier()

    pltpu.async_copy(hp.at[sidx.at[0, 0]], bufs.at[0], gsem)
    pltpu.async_copy(hp.at[sidx.at[0, 1]], bufs.at[1], gsem)
    pltpu.async_copy(hp.at[sidx.at[0, 2]], bufs.at[2], gsem)

    def step(j, carry):
        g = j // GB
        p = j % GB
        sl = g % 2
        b = j % NBUF
        jm = j - 1
        slm = (jm // GB) % 2
        pm = jm % GB
        bm = jm % NBUF
        j2 = j + (NBUF - 1)
        sl2 = (j2 // GB) % 2
        p2 = j2 % GB
        b2 = j2 % NBUF

        pltpu.make_async_copy(hp.at[sidx.at[sl, p]], bufs.at[b], gsem).wait()
        pltpu.async_copy(bufs.at[b], acc.at[didx.at[sl, p]], ssem, add=True)

        @pl.when(j > 0)
        def _wait_prev_scatter():
            pltpu.make_async_copy(
                bufs.at[bm], acc.at[didx.at[slm, pm]], ssem
            ).wait()

        @pl.when(jnp.logical_and(p == 0, g + 1 < NGRP))
        def _prefetch_idx():
            pltpu.async_copy(
                srcr.at[c, s, pl.ds((g + 1) * GB, GB)], sidx.at[1 - sl], isem
            )
            pltpu.async_copy(
                dstr.at[c, s, pl.ds((g + 1) * GB, GB)], didx.at[1 - sl], isem
            )

        @pl.when(jnp.logical_and(p == GB - NBUF, g + 1 < NGRP))
        def _wait_idx():
            pltpu.make_async_copy(
                srcr.at[c, s, pl.ds((g + 1) * GB, GB)], sidx.at[1 - sl], isem
            ).wait()
            pltpu.make_async_copy(
                dstr.at[c, s, pl.ds((g + 1) * GB, GB)], didx.at[1 - sl], isem
            ).wait()

        @pl.when(j2 < CPT)
        def _next_gather():
            pltpu.async_copy(hp.at[sidx.at[sl2, p2]], bufs.at[b2], gsem)

        return carry

    lax.fori_loop(0, CPT, step, 0)
    pltpu.make_async_copy(
        bufs.at[(CPT - 1) % NBUF],
        acc.at[didx.at[((CPT - 1) // GB) % 2, (CPT - 1) % GB]],
        ssem,
    ).wait()
    plsc.subcore_barrier()
    pltpu.sync_copy(acc.at[pl.ds(r0, RPT)], out.at[c, pl.ds(r0, RPT)])


def _tc1_body(xp, w1, deg, hp1, dinv):
    dg = deg[0] + deg[1] + 1.0
    di = lax.rsqrt(dg)
    dinv[...] = di
    t0 = jnp.dot(xp[...], w1[...], preferred_element_type=jnp.float32)
    hp1[...] = t0 * di


def _tc_scale_in(xp, w1, degr):
    return pl.pallas_call(
        _tc1_body,
        grid=(NP // R,),
        in_specs=[
            pl.BlockSpec((R, D), lambda i: (i, 0)),
            pl.BlockSpec((D, D), lambda i: (0, 0)),
            pl.BlockSpec((NC, R, 1), lambda i: (0, i, 0)),
        ],
        out_specs=[
            pl.BlockSpec((R, D), lambda i: (i, 0)),
            pl.BlockSpec((R, 1), lambda i: (i, 0)),
        ],
        out_shape=[
            jax.ShapeDtypeStruct((NP, D), jnp.float32),
            jax.ShapeDtypeStruct((NP, 1), jnp.float32),
        ],
    )(xp, w1, degr)


def _tc_mid_body(agg, hp1, dinv, b1, gamma, beta, w2, hp2, stats):
    p = pl.program_id(0)
    i = pl.program_id(1)
    h = (agg[0] + agg[1] - hp1[...]) * dinv[...] + b1[...]

    @pl.when(p == 0)
    def _accumulate_stats():
        rows = lax.broadcasted_iota(jnp.int32, (R, 1), 0) + i * R
        m = (rows < N).astype(jnp.float32)
        hm = h * m

        @pl.when(i == 0)
        def _():
            stats[...] = jnp.zeros_like(stats)

        stats[0:1, :] += jnp.sum(hm, axis=0, keepdims=True)
        stats[1:2, :] += jnp.sum(hm * h, axis=0, keepdims=True)

    @pl.when(p == 1)
    def _normalize_matmul():
        mean = stats[0:1, :] / N
        var = stats[1:2, :] / N - mean * mean
        inv = lax.rsqrt(var + EPS)
        hb = (h - mean) * inv * gamma[...] + beta[...]
        hb = jnp.maximum(hb, 0.0)
        hp2[...] = (
            jnp.dot(hb, w2[...], preferred_element_type=jnp.float32) * dinv[...]
        )


def _tc_mid(agg1, hp1, dinv, b1, gamma, beta, w2):
    return pl.pallas_call(
        _tc_mid_body,
        grid=(2, NP // R),
        in_specs=[
            pl.BlockSpec((NC, R, D), lambda p, i: (0, i, 0)),
            pl.BlockSpec((R, D), lambda p, i: (i, 0)),
            pl.BlockSpec((R, 1), lambda p, i: (i, 0)),
            pl.BlockSpec((D,), lambda p, i: (0,)),
            pl.BlockSpec((D,), lambda p, i: (0,)),
            pl.BlockSpec((D,), lambda p, i: (0,)),
            pl.BlockSpec((D, D), lambda p, i: (0, 0)),
        ],
        out_specs=pl.BlockSpec((R, D), lambda p, i: (i, 0)),
        out_shape=jax.ShapeDtypeStruct((NP, D), jnp.float32),
        scratch_shapes=[pltpu.VMEM((2, D), jnp.float32)],
    )(agg1, hp1, dinv, b1, gamma, beta, w2)


def _tc4_body(agg, hp2, dinv, b2, batchp, linw, linb, out, sums, cnts):
    i = pl.program_id(0)
    h2 = (agg[0] + agg[1] - hp2[...]) * dinv[...] + b2[...]
    oh = (batchp[...] == lax.broadcasted_iota(jnp.int32, (R, G), 1)).astype(
        jnp.float32
    )

    @pl.when(i == 0)
    def _():
        sums[...] = jnp.zeros_like(sums)
        cnts[...] = jnp.zeros_like(cnts)

    sums[...] += lax.dot_general(
        oh, h2, (((0,), (0,)), ((), ())), preferred_element_type=jnp.float32
    )
    cnts[...] += lax.dot_general(
        oh,
        jnp.ones((R, 1), jnp.float32),
        (((0,), (0,)), ((), ())),
        preferred_element_type=jnp.float32,
    )

    @pl.when(i == pl.num_programs(0) - 1)
    def _():
        pooled = sums[...] / jnp.maximum(cnts[...], 1.0)
        out[...] = (
            jnp.dot(pooled, linw[...], preferred_element_type=jnp.float32)
            + linb[...]
        )


def _tc_final(agg2, hp2, dinv, b2, batchp, lin_w, lin_b):
    return pl.pallas_call(
        _tc4_body,
        grid=(NP // R,),
        in_specs=[
            pl.BlockSpec((NC, R, D), lambda i: (0, i, 0)),
            pl.BlockSpec((R, D), lambda i: (i, 0)),
            pl.BlockSpec((R, 1), lambda i: (i, 0)),
            pl.BlockSpec((D,), lambda i: (0,)),
            pl.BlockSpec((R, 1), lambda i: (i, 0)),
            pl.BlockSpec((D, D), lambda i: (0, 0)),
            pl.BlockSpec((D,), lambda i: (0,)),
        ],
        out_specs=pl.BlockSpec((G, D), lambda i: (0, 0)),
        out_shape=jax.ShapeDtypeStruct((G, D), jnp.float32),
        scratch_shapes=[
            pltpu.VMEM((G, D), jnp.float32),
            pltpu.VMEM((G, 1), jnp.float32),
        ],
    )(agg2, hp2, dinv, b2, batchp, lin_w, lin_b)


def kernel(x, edge_index, batch, W1, b1, W2, b2, bn_gamma, bn_beta, lin_W, lin_b):
    src = edge_index[0].astype(jnp.int32)
    dst = edge_index[1].astype(jnp.int32)
    padi = N + (jnp.arange(EPAD - E, dtype=jnp.int32) % (NP - N))
    srcp = jnp.concatenate([src, padi]).reshape(NC, NS, CPT, CHUNK)
    dstp = jnp.concatenate([dst, padi]).reshape(NC, NS, CPT, CHUNK)
    xp = jnp.zeros((NP, D), jnp.float32).at[:N].set(x)
    batchp = jnp.concatenate(
        [batch.astype(jnp.int32), jnp.full((NP - N,), G, jnp.int32)]
    ).reshape(NP, 1)

    deg = _sc_degree(dstp)
    degr = deg.reshape(NC, NP, 1)
    hp1, dinv = _tc_scale_in(xp, W1, degr)
    agg1 = _sc_aggregate(hp1, srcp, dstp)
    hp2 = _tc_mid(agg1, hp1, dinv, b1, bn_gamma, bn_beta, W2)
    agg2 = _sc_aggregate(hp2, srcp, dstp)
    return _tc_final(agg2, hp2, dinv, b2, batchp, lin_W, lin_b)
